# baseline (device time: 151085 ns/iter reference)
import jax
import jax.numpy as jnp
from jax import lax
from jax.experimental import pallas as pl
from jax.experimental.pallas import tpu as pltpu

N_DEV = 8
B = 2
SQ = 256
D = 768
HQ_PER = 8
DH = 64
SCALE = 0.125


def kernel(x, Wq, Wo, Wk, Wv):
    def body(x_ref, wq_ref, wo_ref, wk_ref, wv_ref, out_ref,
             comm_ref, send_sems, recv_sems):
        my = lax.axis_index("i")
        left = lax.rem(my - 1 + N_DEV, N_DEV)
        right = lax.rem(my + 1, N_DEV)

        barrier_sem = pltpu.get_barrier_semaphore()
        for nbr in (left, right):
            pl.semaphore_signal(
                barrier_sem, inc=1,
                device_id=(nbr,), device_id_type=pl.DeviceIdType.MESH,
            )
        pl.semaphore_wait(barrier_sem, 2)

        kv0 = my * (2 * DH)
        for b in range(B):
            xb = x_ref[b]
            qb = jnp.dot(xb, wq_ref[...],
                         preferred_element_type=jnp.float32)
            kb = jnp.dot(xb, wk_ref[:, pl.ds(kv0, 2 * DH)],
                         preferred_element_type=jnp.float32)
            vb = jnp.dot(xb, wv_ref[:, pl.ds(kv0, 2 * DH)],
                         preferred_element_type=jnp.float32)
            o_parts = []
            for h in range(HQ_PER):
                q_h = qb[:, h * DH:(h + 1) * DH]
                g = h // 4
                k_h = kb[:, g * DH:(g + 1) * DH]
                v_h = vb[:, g * DH:(g + 1) * DH]
                s = lax.dot_general(
                    q_h, k_h, (((1,), (1,)), ((), ())),
                    preferred_element_type=jnp.float32) * SCALE
                m = jnp.max(s, axis=1, keepdims=True)
                p = jnp.exp(s - m)
                l = jnp.sum(p, axis=1, keepdims=True)
                o_h = jnp.dot(p, v_h,
                              preferred_element_type=jnp.float32) / l
                o_parts.append(o_h)
            ob = jnp.concatenate(o_parts, axis=1)
            yb = jnp.dot(ob, wo_ref[...],
                         preferred_element_type=jnp.float32)
            out_ref[b] = yb
            comm_ref[0, pl.ds(b * SQ, SQ), :] = yb

        for h in range(N_DEV - 1):
            rdma = pltpu.make_async_remote_copy(
                src_ref=comm_ref.at[h],
                dst_ref=comm_ref.at[h + 1],
                send_sem=send_sems.at[h],
                recv_sem=recv_sems.at[h + 1],
                device_id=(right,),
                device_id_type=pl.DeviceIdType.MESH,
            )
            rdma.start()
            rdma.wait()
            for b in range(B):
                out_ref[b] = out_ref[b] + comm_ref[h + 1, pl.ds(b * SQ, SQ), :]

    return pl.pallas_call(
        body,
        out_shape=jax.ShapeDtypeStruct((B, SQ, D), jnp.float32),
        in_specs=[pl.BlockSpec(memory_space=pltpu.VMEM)] * 5,
        out_specs=pl.BlockSpec(memory_space=pltpu.VMEM),
        scratch_shapes=[
            pltpu.VMEM((N_DEV, B * SQ, D), jnp.float32),
            pltpu.SemaphoreType.DMA((N_DEV,)),
            pltpu.SemaphoreType.DMA((N_DEV,)),
        ],
        compiler_params=pltpu.CompilerParams(collective_id=0),
    )(x, Wq, Wo, Wk, Wv)


# device time: 55270 ns/iter; 2.7336x vs baseline; 2.7336x over previous
import jax
import jax.numpy as jnp
from jax import lax
from jax.experimental import pallas as pl
from jax.experimental.pallas import tpu as pltpu

N_DEV = 8
B = 2
SQ = 256
D = 768
HQ_PER = 8
DH = 64
SCALE = 0.125
R = B * SQ


def kernel(x, Wq, Wo, Wk, Wv):
    def body(x_ref, wq_ref, wo_ref, wk_ref, wv_ref, out_ref,
             y_ref, recv_ref, send_sems, recv_sems):
        p = lax.axis_index("i")
        bit0 = lax.rem(p, 2)
        bit1 = lax.rem(lax.div(p, 2), 2)
        cz = lax.div(p, 4)
        cx = jnp.bitwise_xor(bit0, bit1)
        cy = bit1
        px = jnp.bitwise_xor(p, 1)
        py = 4 * cz + (3 - lax.rem(p, 4))
        pz = lax.rem(p + 4, N_DEV)

        barrier_sem = pltpu.get_barrier_semaphore()
        for nbr in (px, py, pz):
            pl.semaphore_signal(
                barrier_sem, inc=1,
                device_id=(nbr,), device_id_type=pl.DeviceIdType.MESH,
            )
        pl.semaphore_wait(barrier_sem, 3)

        keep1 = cx * 256
        keep2 = keep1 + cy * 128
        keep3 = keep2 + cz * 64
        send1 = (1 - cx) * 256

        kv0 = p * (2 * DH)

        def compute_batch(b_idx):
            xb = x_ref[b_idx]
            qb = jnp.dot(xb, wq_ref[...],
                         preferred_element_type=jnp.float32)
            kb = jnp.dot(xb, wk_ref[:, pl.ds(kv0, 2 * DH)],
                         preferred_element_type=jnp.float32)
            vb = jnp.dot(xb, wv_ref[:, pl.ds(kv0, 2 * DH)],
                         preferred_element_type=jnp.float32)
            o_parts = []
            for h in range(HQ_PER):
                q_h = qb[:, h * DH:(h + 1) * DH]
                g = h // 4
                k_h = kb[:, g * DH:(g + 1) * DH]
                v_h = vb[:, g * DH:(g + 1) * DH]
                s = lax.dot_general(
                    q_h, k_h, (((1,), (1,)), ((), ())),
                    preferred_element_type=jnp.float32) * SCALE
                m = jnp.max(s, axis=1, keepdims=True)
                pj = jnp.exp(s - m)
                l = jnp.sum(pj, axis=1, keepdims=True)
                o_parts.append(jnp.dot(pj, v_h,
                                       preferred_element_type=jnp.float32) / l)
            ob = jnp.concatenate(o_parts, axis=1)
            yb = jnp.dot(ob, wo_ref[...],
                         preferred_element_type=jnp.float32)
            y_ref[pl.ds(b_idx * SQ, SQ), :] = yb

        compute_batch(1 - cx)
        rdma1 = pltpu.make_async_remote_copy(
            src_ref=y_ref.at[pl.ds(send1, 256)],
            dst_ref=recv_ref.at[pl.ds(0, 256)],
            send_sem=send_sems.at[0],
            recv_sem=recv_sems.at[0],
            device_id=(px,), device_id_type=pl.DeviceIdType.MESH,
        )
        rdma1.start()
        compute_batch(cx)
        rdma1.wait()
        y_ref[pl.ds(keep1, 256), :] = (
            y_ref[pl.ds(keep1, 256), :] + recv_ref[pl.ds(0, 256), :])

        rdma2 = pltpu.make_async_remote_copy(
            src_ref=y_ref.at[pl.ds(keep1 + (1 - cy) * 128, 128)],
            dst_ref=recv_ref.at[pl.ds(256, 128)],
            send_sem=send_sems.at[1],
            recv_sem=recv_sems.at[1],
            device_id=(py,), device_id_type=pl.DeviceIdType.MESH,
        )
        rdma2.start()
        rdma2.wait()
        y_ref[pl.ds(keep2, 128), :] = (
            y_ref[pl.ds(keep2, 128), :] + recv_ref[pl.ds(256, 128), :])

        rdma3 = pltpu.make_async_remote_copy(
            src_ref=y_ref.at[pl.ds(keep2 + (1 - cz) * 64, 64)],
            dst_ref=recv_ref.at[pl.ds(384, 64)],
            send_sem=send_sems.at[2],
            recv_sem=recv_sems.at[2],
            device_id=(pz,), device_id_type=pl.DeviceIdType.MESH,
        )
        rdma3.start()
        rdma3.wait()
        y_ref[pl.ds(keep3, 64), :] = (
            y_ref[pl.ds(keep3, 64), :] + recv_ref[pl.ds(384, 64), :])

        ag4 = pltpu.make_async_remote_copy(
            src_ref=y_ref.at[pl.ds(keep3, 64)],
            dst_ref=y_ref.at[pl.ds(keep3, 64)],
            send_sem=send_sems.at[3], recv_sem=recv_sems.at[3],
            device_id=(pz,), device_id_type=pl.DeviceIdType.MESH,
        )
        ag4.start()
        ag4.wait()
        ag5 = pltpu.make_async_remote_copy(
            src_ref=y_ref.at[pl.ds(keep2, 128)],
            dst_ref=y_ref.at[pl.ds(keep2, 128)],
            send_sem=send_sems.at[4], recv_sem=recv_sems.at[4],
            device_id=(py,), device_id_type=pl.DeviceIdType.MESH,
        )
        ag5.start()
        ag5.wait()
        ag6 = pltpu.make_async_remote_copy(
            src_ref=y_ref.at[pl.ds(keep1, 256)],
            dst_ref=y_ref.at[pl.ds(keep1, 256)],
            send_sem=send_sems.at[5], recv_sem=recv_sems.at[5],
            device_id=(px,), device_id_type=pl.DeviceIdType.MESH,
        )
        ag6.start()
        ag6.wait()

        out_ref[0] = y_ref[pl.ds(0, SQ), :]
        out_ref[1] = y_ref[pl.ds(SQ, SQ), :]

    return pl.pallas_call(
        body,
        out_shape=jax.ShapeDtypeStruct((B, SQ, D), jnp.float32),
        in_specs=[pl.BlockSpec(memory_space=pltpu.VMEM)] * 5,
        out_specs=pl.BlockSpec(memory_space=pltpu.VMEM),
        scratch_shapes=[
            pltpu.VMEM((R, D), jnp.float32),
            pltpu.VMEM((448, D), jnp.float32),
            pltpu.SemaphoreType.DMA((6,)),
            pltpu.SemaphoreType.DMA((6,)),
        ],
        compiler_params=pltpu.CompilerParams(collective_id=0),
    )(x, Wq, Wo, Wk, Wv)


# device time: 44274 ns/iter; 3.4125x vs baseline; 1.2484x over previous
import jax
import jax.numpy as jnp
from jax import lax
from jax.experimental import pallas as pl
from jax.experimental.pallas import tpu as pltpu

N_DEV = 8
B = 2
SQ = 256
D = 768
HQ_PER = 8
DH = 64
SCALE = 0.125
R = B * SQ
SEG = R // N_DEV


def kernel(x, Wq, Wo, Wk, Wv):
    def body(x_ref, wq_ref, wo_ref, wk_ref, wv_ref, out_ref,
             y_ref, recv_ref, send_sems, recv_sems):
        p = lax.axis_index("i")
        bit0 = lax.rem(p, 2)
        bit1 = lax.rem(lax.div(p, 2), 2)
        cz = lax.div(p, 4)
        cx = jnp.bitwise_xor(bit0, bit1)
        cy = bit1

        peers = []
        for m in range(1, N_DEV):
            qx = jnp.bitwise_xor(cx, m & 1)
            qy = jnp.bitwise_xor(cy, (m >> 1) & 1)
            qz = jnp.bitwise_xor(cz, (m >> 2) & 1)
            q = 4 * qz + 2 * qy + jnp.bitwise_xor(qx, qy)
            qbase = 256 * qx + 128 * qy + 64 * qz
            peers.append((m, q, qbase, m & 1))
        my_base = 256 * cx + 128 * cy + 64 * cz

        barrier_sem = pltpu.get_barrier_semaphore()
        for (_, q, _, _) in peers:
            pl.semaphore_signal(
                barrier_sem, inc=1,
                device_id=(q,), device_id_type=pl.DeviceIdType.MESH,
            )
        pl.semaphore_wait(barrier_sem, N_DEV - 1)

        kv0 = p * (2 * DH)

        def compute_batch(b_idx):
            xb = x_ref[b_idx]
            qb = jnp.dot(xb, wq_ref[...],
                         preferred_element_type=jnp.float32)
            kb = jnp.dot(xb, wk_ref[:, pl.ds(kv0, 2 * DH)],
                         preferred_element_type=jnp.float32)
            vb = jnp.dot(xb, wv_ref[:, pl.ds(kv0, 2 * DH)],
                         preferred_element_type=jnp.float32)
            o_parts = []
            for h in range(HQ_PER):
                q_h = qb[:, h * DH:(h + 1) * DH]
                g = h // 4
                k_h = kb[:, g * DH:(g + 1) * DH]
                v_h = vb[:, g * DH:(g + 1) * DH]
                s = lax.dot_general(
                    q_h, k_h, (((1,), (1,)), ((), ())),
                    preferred_element_type=jnp.float32) * SCALE
                mx = jnp.max(s, axis=1, keepdims=True)
                pj = jnp.exp(s - mx)
                l = jnp.sum(pj, axis=1, keepdims=True)
                o_parts.append(jnp.dot(pj, v_h,
                                       preferred_element_type=jnp.float32) / l)
            ob = jnp.concatenate(o_parts, axis=1)
            yb = jnp.dot(ob, wo_ref[...],
                         preferred_element_type=jnp.float32)
            y_ref[pl.ds(b_idx * SQ, SQ), :] = yb

        rs = []
        for (m, q, qbase, xflip) in peers:
            rs.append(pltpu.make_async_remote_copy(
                src_ref=y_ref.at[pl.ds(qbase, SEG)],
                dst_ref=recv_ref.at[pl.ds((m - 1) * SEG, SEG)],
                send_sem=send_sems.at[m - 1],
                recv_sem=recv_sems.at[m - 1],
                device_id=(q,), device_id_type=pl.DeviceIdType.MESH,
            ))

        compute_batch(0)
        for (m, q, qbase, xflip), r in zip(peers, rs):
            in_b0 = jnp.bitwise_xor(cx, xflip) == 0

            @pl.when(in_b0)
            def _(r=r):
                r.start()

        compute_batch(1)
        for (m, q, qbase, xflip), r in zip(peers, rs):
            in_b1 = jnp.bitwise_xor(cx, xflip) == 1

            @pl.when(in_b1)
            def _(r=r):
                r.start()

        for r in rs:
            r.wait()

        acc = recv_ref[pl.ds(0, SEG), :]
        for m in range(2, N_DEV):
            acc = acc + recv_ref[pl.ds((m - 1) * SEG, SEG), :]
        y_ref[pl.ds(my_base, SEG), :] = y_ref[pl.ds(my_base, SEG), :] + acc

        ag = []
        for (m, q, qbase, xflip) in peers:
            a = pltpu.make_async_remote_copy(
                src_ref=y_ref.at[pl.ds(my_base, SEG)],
                dst_ref=y_ref.at[pl.ds(my_base, SEG)],
                send_sem=send_sems.at[N_DEV - 1 + m - 1],
                recv_sem=recv_sems.at[N_DEV - 1 + m - 1],
                device_id=(q,), device_id_type=pl.DeviceIdType.MESH,
            )
            a.start()
            ag.append(a)
        for a in ag:
            a.wait()

        out_ref[0] = y_ref[pl.ds(0, SQ), :]
        out_ref[1] = y_ref[pl.ds(SQ, SQ), :]

    return pl.pallas_call(
        body,
        out_shape=jax.ShapeDtypeStruct((B, SQ, D), jnp.float32),
        in_specs=[pl.BlockSpec(memory_space=pltpu.VMEM)] * 5,
        out_specs=pl.BlockSpec(memory_space=pltpu.VMEM),
        scratch_shapes=[
            pltpu.VMEM((R, D), jnp.float32),
            pltpu.VMEM(((N_DEV - 1) * SEG, D), jnp.float32),
            pltpu.SemaphoreType.DMA((2 * (N_DEV - 1),)),
            pltpu.SemaphoreType.DMA((2 * (N_DEV - 1),)),
        ],
        compiler_params=pltpu.CompilerParams(collective_id=0),
    )(x, Wq, Wo, Wk, Wv)


# device time: 25921 ns/iter; 5.8287x vs baseline; 1.7080x over previous
import jax
import jax.numpy as jnp
from jax import lax
from jax.experimental import pallas as pl
from jax.experimental.pallas import tpu as pltpu

N_DEV = 8
B = 2
SQ = 256
D = 768
HQ_PER = 8
DH = 64
SCALE = 0.125
R = B * SQ
SEG = R // N_DEV


def kernel(x, Wq, Wo, Wk, Wv):
    bf = jnp.bfloat16
    idx = lax.axis_index("i")
    x_b = x.astype(bf)
    wq_b = Wq.astype(bf)
    wo_b = Wo.astype(bf)
    wk_b = lax.dynamic_slice(Wk, (0, idx * (2 * DH)), (D, 2 * DH)).astype(bf)
    wv_b = lax.dynamic_slice(Wv, (0, idx * (2 * DH)), (D, 2 * DH)).astype(bf)

    def body(x_hbm, wq_hbm, wo_hbm, wk_hbm, wv_hbm, out_ref,
             y_ref, recv_ref, x_ref, wq_ref, wo_ref, wk_sl, wv_sl,
             copy_sems, send_sems, recv_sems):
        p = lax.axis_index("i")
        bit0 = lax.rem(p, 2)
        bit1 = lax.rem(lax.div(p, 2), 2)
        cz = lax.div(p, 4)
        cx = jnp.bitwise_xor(bit0, bit1)
        cy = bit1

        cps = [
            pltpu.make_async_copy(x_hbm, x_ref, copy_sems.at[0]),
            pltpu.make_async_copy(wq_hbm, wq_ref, copy_sems.at[1]),
            pltpu.make_async_copy(wo_hbm, wo_ref, copy_sems.at[2]),
            pltpu.make_async_copy(wk_hbm, wk_sl, copy_sems.at[3]),
            pltpu.make_async_copy(wv_hbm, wv_sl, copy_sems.at[4]),
        ]
        for c in cps:
            c.start()

        peers = []
        for m in range(1, N_DEV):
            qx = jnp.bitwise_xor(cx, m & 1)
            qy = jnp.bitwise_xor(cy, (m >> 1) & 1)
            qz = jnp.bitwise_xor(cz, (m >> 2) & 1)
            q = 4 * qz + 2 * qy + jnp.bitwise_xor(qx, qy)
            qbase = 256 * qx + 128 * qy + 64 * qz
            peers.append((m, q, qbase, m & 1))
        my_base = 256 * cx + 128 * cy + 64 * cz

        barrier_sem = pltpu.get_barrier_semaphore()
        for (_, q, _, _) in peers:
            pl.semaphore_signal(
                barrier_sem, inc=1,
                device_id=(q,), device_id_type=pl.DeviceIdType.MESH,
            )

        for c in cps:
            c.wait()

        def compute_batch(b_idx):
            xb = x_ref[b_idx]
            qb = jnp.dot(xb, wq_ref[...],
                         preferred_element_type=jnp.float32)
            kb = jnp.dot(xb, wk_sl[...],
                         preferred_element_type=jnp.float32)
            vb = jnp.dot(xb, wv_sl[...],
                         preferred_element_type=jnp.float32)
            o_parts = []
            for h in range(HQ_PER):
                q_h = qb[:, h * DH:(h + 1) * DH].astype(bf)
                g = h // 4
                k_h = kb[:, g * DH:(g + 1) * DH].astype(bf)
                v_h = vb[:, g * DH:(g + 1) * DH].astype(bf)
                s = lax.dot_general(
                    q_h, k_h, (((1,), (1,)), ((), ())),
                    preferred_element_type=jnp.float32) * SCALE
                mx = jnp.max(s, axis=1, keepdims=True)
                pj = jnp.exp(s - mx)
                l = jnp.sum(pj, axis=1, keepdims=True)
                o_h = jnp.dot(pj.astype(bf), v_h,
                              preferred_element_type=jnp.float32) / l
                o_parts.append(o_h.astype(bf))
            ob = jnp.concatenate(o_parts, axis=1)
            yb = jnp.dot(ob, wo_ref[...],
                         preferred_element_type=jnp.float32)
            y_ref[pl.ds(b_idx * SQ, SQ), :] = yb.astype(bf)

        rs = []
        for (m, q, qbase, xflip) in peers:
            rs.append(pltpu.make_async_remote_copy(
                src_ref=y_ref.at[pl.ds(qbase, SEG)],
                dst_ref=recv_ref.at[pl.ds((m - 1) * SEG, SEG)],
                send_sem=send_sems.at[m - 1],
                recv_sem=recv_sems.at[m - 1],
                device_id=(q,), device_id_type=pl.DeviceIdType.MESH,
            ))

        b_first = cz
        compute_batch(b_first)
        pl.semaphore_wait(barrier_sem, N_DEV - 1)
        for (m, q, qbase, xflip), r in zip(peers, rs):
            @pl.when(lax.div(qbase, 256) == b_first)
            def _(r=r):
                r.start()

        compute_batch(1 - b_first)
        for (m, q, qbase, xflip), r in zip(peers, rs):
            @pl.when(lax.div(qbase, 256) == 1 - b_first)
            def _(r=r):
                r.start()

        for r in rs:
            r.wait()

        acc = y_ref[pl.ds(my_base, SEG), :].astype(jnp.float32)
        for m in range(1, N_DEV):
            acc = acc + recv_ref[pl.ds((m - 1) * SEG, SEG), :].astype(
                jnp.float32)
        y_ref[pl.ds(my_base, SEG), :] = acc.astype(bf)

        ag = []
        for (m, q, qbase, xflip) in sorted(peers, key=lambda t: -t[0]):
            a = pltpu.make_async_remote_copy(
                src_ref=y_ref.at[pl.ds(my_base, SEG)],
                dst_ref=y_ref.at[pl.ds(my_base, SEG)],
                send_sem=send_sems.at[N_DEV - 1 + m - 1],
                recv_sem=recv_sems.at[N_DEV - 1 + m - 1],
                device_id=(q,), device_id_type=pl.DeviceIdType.MESH,
            )
            a.start()
            ag.append(a)
        for a in ag:
            a.wait_recv()
        for a in ag:
            a.wait_send()

        out_ref[0] = y_ref[pl.ds(0, SQ), :]
        out_ref[1] = y_ref[pl.ds(SQ, SQ), :]

    return pl.pallas_call(
        body,
        out_shape=jax.ShapeDtypeStruct((B, SQ, D), jnp.bfloat16),
        in_specs=[pl.BlockSpec(memory_space=pltpu.MemorySpace.HBM)] * 5,
        out_specs=pl.BlockSpec(memory_space=pltpu.VMEM),
        scratch_shapes=[
            pltpu.VMEM((R, D), jnp.bfloat16),
            pltpu.VMEM(((N_DEV - 1) * SEG, D), jnp.bfloat16),
            pltpu.VMEM((B, SQ, D), jnp.bfloat16),
            pltpu.VMEM((D, HQ_PER * DH), jnp.bfloat16),
            pltpu.VMEM((HQ_PER * DH, D), jnp.bfloat16),
            pltpu.VMEM((D, 2 * DH), jnp.bfloat16),
            pltpu.VMEM((D, 2 * DH), jnp.bfloat16),
            pltpu.SemaphoreType.DMA((5,)),
            pltpu.SemaphoreType.DMA((2 * (N_DEV - 1),)),
            pltpu.SemaphoreType.DMA((2 * (N_DEV - 1),)),
        ],
        compiler_params=pltpu.CompilerParams(collective_id=0),
    )(x_b, wq_b, wo_b, wk_b, wv_b)
